# baseline (speedup 1.0000x reference)
import jax
import jax.numpy as jnp
from jax.experimental import pallas as pl
from jax.experimental.pallas import tpu as pltpu

_BM_A = 200
_BM_B = 1000
_S2_SCALE = 0.125


def _s1_body(z_ref, w1_ref, out_ref):
    out_ref[...] = jax.lax.dot(
        z_ref[...], w1_ref[...], preferred_element_type=jnp.float32
    )


def _layer1_body(adjt_ref, adjb_ref, s1_ref, b1_ref, w2_ref, s2_ref, q8_ref):
    s1 = s1_ref[...]
    w2 = w2_ref[...]
    b1 = b1_ref[...]
    for k, a_ref in enumerate((adjt_ref, adjb_ref)):
        a = a_ref[...]
        h = jax.lax.dot(a, s1, preferred_element_type=jnp.float32)
        h = jnp.maximum(h + b1, 0.0)
        s2 = jax.lax.dot(h, w2, preferred_element_type=jnp.float32)
        rows = pl.ds(k * _BM_A, _BM_A)
        s2_ref[rows, :] = (s2 * _S2_SCALE).astype(jnp.float8_e4m3fn)
        q8_ref[rows, :] = a.astype(jnp.float8_e4m3fn)


def _layer2_body(q8t_ref, q8b_ref, s2_ref, b2_ref, out_ref):
    s2 = s2_ref[...]
    b2 = b2_ref[...]
    for k, q_ref in enumerate((q8t_ref, q8b_ref)):
        acc = jax.lax.dot(q_ref[...], s2, preferred_element_type=jnp.float32)
        out_ref[pl.ds(k * _BM_B, _BM_B), :] = acc * (1.0 / _S2_SCALE) + b2


def kernel(z, adj, W1, b1, W2, b2):
    n, _ = z.shape
    m = adj.shape[0]
    h_dim = W1.shape[1]
    f_dim = W2.shape[1]
    b1r = b1.reshape(1, h_dim)
    b2r = b2.reshape(1, f_dim)

    s1 = pl.pallas_call(
        _s1_body,
        out_shape=jax.ShapeDtypeStruct((n, h_dim), jnp.float32),
    )(z, W1)

    parallel = pltpu.CompilerParams(dimension_semantics=("parallel",))

    s2q, q8 = pl.pallas_call(
        _layer1_body,
        grid=(pl.cdiv(m, 2 * _BM_A),),
        in_specs=[
            pl.BlockSpec((_BM_A, n), lambda i: (2 * i, 0)),
            pl.BlockSpec((_BM_A, n), lambda i: (2 * i + 1, 0)),
            pl.BlockSpec((n, h_dim), lambda i: (0, 0)),
            pl.BlockSpec((1, h_dim), lambda i: (0, 0)),
            pl.BlockSpec((h_dim, f_dim), lambda i: (0, 0)),
        ],
        out_specs=[
            pl.BlockSpec((2 * _BM_A, f_dim), lambda i: (i, 0)),
            pl.BlockSpec((2 * _BM_A, n), lambda i: (i, 0)),
        ],
        out_shape=[
            jax.ShapeDtypeStruct((m, f_dim), jnp.float8_e4m3fn),
            jax.ShapeDtypeStruct((m, n), jnp.float8_e4m3fn),
        ],
        compiler_params=parallel,
    )(adj, adj, s1, b1r, W2)

    out = pl.pallas_call(
        _layer2_body,
        grid=(pl.cdiv(m, 2 * _BM_B),),
        in_specs=[
            pl.BlockSpec((_BM_B, n), lambda i: (2 * i, 0)),
            pl.BlockSpec((_BM_B, n), lambda i: (2 * i + 1, 0)),
            pl.BlockSpec((n, f_dim), lambda i: (0, 0)),
            pl.BlockSpec((1, f_dim), lambda i: (0, 0)),
        ],
        out_specs=pl.BlockSpec((2 * _BM_B, f_dim), lambda i: (i, 0)),
        out_shape=jax.ShapeDtypeStruct((m, f_dim), jnp.float32),
        compiler_params=parallel,
    )(q8, q8, s2q, b2r)
    return out


# speedup vs baseline: 1.0807x; 1.0807x over previous
import jax
import jax.numpy as jnp
from jax.experimental import pallas as pl
from jax.experimental.pallas import tpu as pltpu

_BM_A = 400
_BM_B = 1000
_S2_SCALE = 0.125


def _layer1_body(z_ref, w1_ref, adj_ref, b1_ref, w2_ref,
                 s2_ref, q8_ref, s1_ref):
    @pl.when(pl.program_id(0) == 0)
    def _():
        s1_ref[...] = jax.lax.dot(
            z_ref[...], w1_ref[...], preferred_element_type=jnp.float32
        )

    a = adj_ref[...]
    h = jax.lax.dot(a, s1_ref[...], preferred_element_type=jnp.float32)
    h = jnp.maximum(h + b1_ref[...], 0.0)
    s2 = jax.lax.dot(h, w2_ref[...], preferred_element_type=jnp.float32)
    s2_ref[...] = (s2 * _S2_SCALE).astype(jnp.float8_e4m3fn)
    q8_ref[...] = a.astype(jnp.float8_e4m3fn)


def _layer2_body(q8_ref, s2_ref, b2_ref, out_ref):
    acc = jax.lax.dot(
        q8_ref[...], s2_ref[...], preferred_element_type=jnp.float32
    )
    out_ref[...] = acc * (1.0 / _S2_SCALE) + b2_ref[...]


def kernel(z, adj, W1, b1, W2, b2):
    n, _ = z.shape
    m = adj.shape[0]
    h_dim = W1.shape[1]
    f_dim = W2.shape[1]
    b1r = b1.reshape(1, h_dim)
    b2r = b2.reshape(1, f_dim)

    s2q, q8 = pl.pallas_call(
        _layer1_body,
        grid=(pl.cdiv(m, _BM_A),),
        in_specs=[
            pl.BlockSpec((n, h_dim), lambda i: (0, 0)),
            pl.BlockSpec((h_dim, h_dim), lambda i: (0, 0)),
            pl.BlockSpec((_BM_A, n), lambda i: (i, 0)),
            pl.BlockSpec((1, h_dim), lambda i: (0, 0)),
            pl.BlockSpec((h_dim, f_dim), lambda i: (0, 0)),
        ],
        out_specs=[
            pl.BlockSpec((_BM_A, f_dim), lambda i: (i, 0)),
            pl.BlockSpec((_BM_A, n), lambda i: (i, 0)),
        ],
        out_shape=[
            jax.ShapeDtypeStruct((m, f_dim), jnp.float8_e4m3fn),
            jax.ShapeDtypeStruct((m, n), jnp.float8_e4m3fn),
        ],
        scratch_shapes=[pltpu.VMEM((n, h_dim), jnp.float32)],
    )(z, W1, adj, b1r, W2)

    out = pl.pallas_call(
        _layer2_body,
        grid=(pl.cdiv(m, _BM_B),),
        in_specs=[
            pl.BlockSpec((_BM_B, n), lambda i: (i, 0)),
            pl.BlockSpec((n, f_dim), lambda i: (0, 0)),
            pl.BlockSpec((1, f_dim), lambda i: (0, 0)),
        ],
        out_specs=pl.BlockSpec((_BM_B, f_dim), lambda i: (i, 0)),
        out_shape=jax.ShapeDtypeStruct((m, f_dim), jnp.float32),
        compiler_params=pltpu.CompilerParams(
            dimension_semantics=("parallel",)
        ),
    )(q8, s2q, b2r)
    return out
